# baseline (device time: 53065 ns/iter reference)
import jax
import jax.numpy as jnp
from jax import lax
from jax.experimental import pallas as pl
from jax.experimental.pallas import tpu as pltpu

N_DEV = 8
B = 2
S = 512
H = 8
D = 64
WIN = 128
E = S + 2 * WIN


def kernel(x, Wq, K_ext, V_ext, Wo):
    def body(x_ref, wq_ref, k_ref, v_ref, wo_ref, out_ref,
             kbuf, vbuf, send_sems, recv_sems):
        my = lax.axis_index("i")
        left = lax.rem(my + (N_DEV - 1), N_DEV)
        right = lax.rem(my + 1, N_DEV)

        barrier = pltpu.get_barrier_semaphore()
        for nbr in (left, right):
            pl.semaphore_signal(barrier, inc=1, device_id=(nbr,),
                                device_id_type=pl.DeviceIdType.MESH)
        pl.semaphore_wait(barrier, 2)

        rdmas = []
        for idx, (src_ref, dst_buf) in enumerate([(k_ref, kbuf), (v_ref, vbuf)]):
            r_l = pltpu.make_async_remote_copy(
                src_ref=src_ref.at[:, pl.ds(0, WIN)],
                dst_ref=dst_buf.at[:, pl.ds(WIN + S, WIN)],
                send_sem=send_sems.at[2 * idx],
                recv_sem=recv_sems.at[2 * idx],
                device_id=(left,), device_id_type=pl.DeviceIdType.MESH)
            r_r = pltpu.make_async_remote_copy(
                src_ref=src_ref.at[:, pl.ds(S - WIN, WIN)],
                dst_ref=dst_buf.at[:, pl.ds(0, WIN)],
                send_sem=send_sems.at[2 * idx + 1],
                recv_sem=recv_sems.at[2 * idx + 1],
                device_id=(right,), device_id_type=pl.DeviceIdType.MESH)
            r_l.start()
            r_r.start()
            rdmas += [r_l, r_r]

        kbuf[:, WIN:WIN + S] = k_ref[...]
        vbuf[:, WIN:WIN + S] = v_ref[...]

        wq = wq_ref[...]
        q = [jnp.dot(x_ref[b], wq, preferred_element_type=jnp.float32)
             for b in range(B)]

        i_loc = lax.broadcasted_iota(jnp.int32, (S, E), 0)
        j_ext = lax.broadcasted_iota(jnp.int32, (S, E), 1)
        j_glob = j_ext + my * S - WIN
        mask = ((j_ext >= i_loc) & (j_ext <= i_loc + 2 * WIN)
                & (j_glob >= 0) & (j_glob < N_DEV * S))

        for r in rdmas:
            r.wait_recv()

        wo = wo_ref[...]
        for b in range(B):
            ctxs = []
            for h in range(H):
                qh = q[b][:, h * D:(h + 1) * D]
                kh = kbuf[b, :, h, :]
                s = lax.dot_general(
                    qh, kh, (((1,), (1,)), ((), ())),
                    preferred_element_type=jnp.float32) * 0.125
                s = jnp.where(mask, s, -1e9)
                m = jnp.max(s, axis=1, keepdims=True)
                w = jnp.exp(s - m)
                w = w / jnp.sum(w, axis=1, keepdims=True)
                ctxs.append(jnp.dot(w, vbuf[b, :, h, :],
                                    preferred_element_type=jnp.float32))
            ctx = jnp.concatenate(ctxs, axis=1)
            out_ref[b] = jnp.dot(ctx, wo, preferred_element_type=jnp.float32)

        for r in rdmas:
            r.wait_send()

    return pl.pallas_call(
        body,
        out_shape=jax.ShapeDtypeStruct((B, S, 768), jnp.float32),
        in_specs=[pl.BlockSpec(memory_space=pltpu.VMEM)] * 5,
        out_specs=pl.BlockSpec(memory_space=pltpu.VMEM),
        scratch_shapes=[
            pltpu.VMEM((B, E, H, D), jnp.float32),
            pltpu.VMEM((B, E, H, D), jnp.float32),
            pltpu.SemaphoreType.DMA((4,)),
            pltpu.SemaphoreType.DMA((4,)),
        ],
        compiler_params=pltpu.CompilerParams(collective_id=0),
    )(x, Wq, K_ext, V_ext, Wo)


# device time: 50774 ns/iter; 1.0451x vs baseline; 1.0451x over previous
import jax
import jax.numpy as jnp
from jax import lax
from jax.experimental import pallas as pl
from jax.experimental.pallas import tpu as pltpu

N_DEV = 8
B = 2
S = 512
H = 8
D = 64
WIN = 128
E = S + 2 * WIN


def kernel(x, Wq, K_ext, V_ext, Wo):
    def body(x_ref, wq_ref, k_ref, v_ref, wo_ref, out_ref,
             kbuf, vbuf, send_sems, recv_sems):
        my = lax.axis_index("i")
        left = lax.rem(my + (N_DEV - 1), N_DEV)
        right = lax.rem(my + 1, N_DEV)

        barrier = pltpu.get_barrier_semaphore()
        for nbr in (left, right):
            pl.semaphore_signal(barrier, inc=1, device_id=(nbr,),
                                device_id_type=pl.DeviceIdType.MESH)
        pl.semaphore_wait(barrier, 2)

        rdmas = []
        for idx, (src_ref, dst_buf) in enumerate([(k_ref, kbuf), (v_ref, vbuf)]):
            r_l = pltpu.make_async_remote_copy(
                src_ref=src_ref.at[:, pl.ds(0, WIN)],
                dst_ref=dst_buf.at[:, pl.ds(WIN + S, WIN)],
                send_sem=send_sems.at[2 * idx],
                recv_sem=recv_sems.at[2 * idx],
                device_id=(left,), device_id_type=pl.DeviceIdType.MESH)
            r_r = pltpu.make_async_remote_copy(
                src_ref=src_ref.at[:, pl.ds(S - WIN, WIN)],
                dst_ref=dst_buf.at[:, pl.ds(0, WIN)],
                send_sem=send_sems.at[2 * idx + 1],
                recv_sem=recv_sems.at[2 * idx + 1],
                device_id=(right,), device_id_type=pl.DeviceIdType.MESH)
            r_l.start()
            r_r.start()
            rdmas += [r_l, r_r]

        kbuf[:, WIN:WIN + S] = k_ref[...]
        vbuf[:, WIN:WIN + S] = v_ref[...]

        wq = wq_ref[...]
        q = [jnp.dot(x_ref[b], wq, preferred_element_type=jnp.float32)
             for b in range(B)]

        QB = 128
        KB = QB + 2 * WIN
        NQB = S // QB
        ii = lax.broadcasted_iota(jnp.int32, (QB, KB), 0)
        jj = lax.broadcasted_iota(jnp.int32, (QB, KB), 1)
        band = (jj >= ii) & (jj <= ii + 2 * WIN)
        masks = []
        for t in range(NQB):
            j_glob = jj + t * QB + my * S - WIN
            masks.append(band & (j_glob >= 0) & (j_glob < N_DEV * S))

        for r in rdmas:
            r.wait_recv()

        wo = wo_ref[...]
        for b in range(B):
            cblk = [[] for _ in range(NQB)]
            for h in range(H):
                kh = kbuf[b, :, h, :]
                vh = vbuf[b, :, h, :]
                for t in range(NQB):
                    qt = q[b][t * QB:(t + 1) * QB, h * D:(h + 1) * D]
                    kt = kh[t * QB:t * QB + KB, :]
                    s = lax.dot_general(
                        qt, kt, (((1,), (1,)), ((), ())),
                        preferred_element_type=jnp.float32) * 0.125
                    w = jnp.exp(jnp.where(masks[t], s, -1e9))
                    denom = jnp.sum(w, axis=1, keepdims=True)
                    c = jnp.dot(w, vh[t * QB:t * QB + KB, :],
                                preferred_element_type=jnp.float32)
                    cblk[t].append(c / denom)
            for t in range(NQB):
                ctx = jnp.concatenate(cblk[t], axis=1)
                out_ref[b, t * QB:(t + 1) * QB] = jnp.dot(
                    ctx, wo, preferred_element_type=jnp.float32)

        for r in rdmas:
            r.wait_send()

    return pl.pallas_call(
        body,
        out_shape=jax.ShapeDtypeStruct((B, S, 768), jnp.float32),
        in_specs=[pl.BlockSpec(memory_space=pltpu.VMEM)] * 5,
        out_specs=pl.BlockSpec(memory_space=pltpu.VMEM),
        scratch_shapes=[
            pltpu.VMEM((B, E, H, D), jnp.float32),
            pltpu.VMEM((B, E, H, D), jnp.float32),
            pltpu.SemaphoreType.DMA((4,)),
            pltpu.SemaphoreType.DMA((4,)),
        ],
        compiler_params=pltpu.CompilerParams(collective_id=0),
    )(x, Wq, K_ext, V_ext, Wo)


# device time: 42456 ns/iter; 1.2499x vs baseline; 1.1959x over previous
import jax
import jax.numpy as jnp
from jax import lax
from jax.experimental import pallas as pl
from jax.experimental.pallas import tpu as pltpu

N_DEV = 8
B = 2
S = 512
H = 8
D = 64
WIN = 128
E = S + 2 * WIN
BF = jnp.bfloat16
F32 = jnp.float32


def kernel(x, Wq, K_ext, V_ext, Wo):
    def body(x_ref, wq_ref, k_ref, v_ref, wo_ref, out_ref,
             kbuf, vbuf, ksend, vsend, send_sems, recv_sems):
        my = lax.axis_index("i")
        left = lax.rem(my + (N_DEV - 1), N_DEV)
        right = lax.rem(my + 1, N_DEV)

        barrier = pltpu.get_barrier_semaphore()
        for nbr in (left, right):
            pl.semaphore_signal(barrier, inc=1, device_id=(nbr,),
                                device_id_type=pl.DeviceIdType.MESH)
        pl.semaphore_wait(barrier, 2)

        ksend[0] = k_ref[:, :WIN].astype(BF)
        ksend[1] = k_ref[:, S - WIN:].astype(BF)
        vsend[0] = v_ref[:, :WIN].astype(BF)
        vsend[1] = v_ref[:, S - WIN:].astype(BF)

        rdmas = []
        for idx, (sbuf, dst_buf) in enumerate([(ksend, kbuf), (vsend, vbuf)]):
            r_l = pltpu.make_async_remote_copy(
                src_ref=sbuf.at[0],
                dst_ref=dst_buf.at[:, pl.ds(WIN + S, WIN)],
                send_sem=send_sems.at[2 * idx],
                recv_sem=recv_sems.at[2 * idx],
                device_id=(left,), device_id_type=pl.DeviceIdType.MESH)
            r_r = pltpu.make_async_remote_copy(
                src_ref=sbuf.at[1],
                dst_ref=dst_buf.at[:, pl.ds(0, WIN)],
                send_sem=send_sems.at[2 * idx + 1],
                recv_sem=recv_sems.at[2 * idx + 1],
                device_id=(right,), device_id_type=pl.DeviceIdType.MESH)
            r_l.start()
            r_r.start()
            rdmas += [r_l, r_r]

        kbuf[:, WIN:WIN + S] = k_ref[...].astype(BF)
        vbuf[:, WIN:WIN + S] = v_ref[...].astype(BF)

        wq = wq_ref[...].astype(BF)
        q = [jnp.dot(x_ref[b].astype(BF), wq,
                     preferred_element_type=F32).astype(BF)
             for b in range(B)]

        QB = 128
        KB = QB + 2 * WIN
        NQB = S // QB
        ii = lax.broadcasted_iota(jnp.int32, (QB, KB), 0)
        jj = lax.broadcasted_iota(jnp.int32, (QB, KB), 1)
        band = (jj >= ii) & (jj <= ii + 2 * WIN)
        masks = []
        for t in range(NQB):
            j_glob = jj + t * QB + my * S - WIN
            masks.append(band & (j_glob >= 0) & (j_glob < N_DEV * S))

        for r in rdmas:
            r.wait_recv()

        wo = wo_ref[...].astype(BF)
        for b in range(B):
            cblk = [[] for _ in range(NQB)]
            for h in range(H):
                kh = kbuf[b, :, h, :]
                vh = vbuf[b, :, h, :]
                for t in range(NQB):
                    qt = q[b][t * QB:(t + 1) * QB, h * D:(h + 1) * D]
                    kt = kh[t * QB:t * QB + KB, :]
                    s = lax.dot_general(
                        qt, kt, (((1,), (1,)), ((), ())),
                        preferred_element_type=F32) * 0.125
                    w = jnp.exp(jnp.where(masks[t], s, -1e9))
                    denom = jnp.sum(w, axis=1, keepdims=True)
                    c = jnp.dot(w.astype(BF), vh[t * QB:t * QB + KB, :],
                                preferred_element_type=F32)
                    cblk[t].append(c / denom)
            for t in range(NQB):
                ctx = jnp.concatenate(cblk[t], axis=1).astype(BF)
                out_ref[b, t * QB:(t + 1) * QB] = jnp.dot(
                    ctx, wo, preferred_element_type=F32)

        for r in rdmas:
            r.wait_send()

    return pl.pallas_call(
        body,
        out_shape=jax.ShapeDtypeStruct((B, S, 768), F32),
        in_specs=[pl.BlockSpec(memory_space=pltpu.VMEM)] * 5,
        out_specs=pl.BlockSpec(memory_space=pltpu.VMEM),
        scratch_shapes=[
            pltpu.VMEM((B, E, H, D), BF),
            pltpu.VMEM((B, E, H, D), BF),
            pltpu.VMEM((2, B, WIN, H, D), BF),
            pltpu.VMEM((2, B, WIN, H, D), BF),
            pltpu.SemaphoreType.DMA((4,)),
            pltpu.SemaphoreType.DMA((4,)),
        ],
        compiler_params=pltpu.CompilerParams(collective_id=0),
    )(x, Wq, K_ext, V_ext, Wo)


# device time: 34234 ns/iter; 1.5501x vs baseline; 1.2402x over previous
import jax
import jax.numpy as jnp
from jax import lax
from jax.experimental import pallas as pl
from jax.experimental.pallas import tpu as pltpu

N_DEV = 8
B = 2
S = 512
H = 8
D = 64
WIN = 128
E = S + 2 * WIN
QB = 128
KB = QB + 2 * WIN
NQB = S // QB
BF = jnp.bfloat16
F32 = jnp.float32


def kernel(x, Wq, K_ext, V_ext, Wo):
    x_bf = x.astype(BF)
    wq_bf = (Wq * 0.125).astype(BF)
    wo_bf = Wo.astype(BF)
    k_bf = jnp.transpose(K_ext, (0, 2, 1, 3)).astype(BF)
    v_bf = jnp.transpose(V_ext, (0, 2, 1, 3)).astype(BF)

    def body(x_ref, wq_ref, k_ref, v_ref, wo_ref, out_ref,
             kbuf, vbuf, ksend, vsend, send_sems, recv_sems):
        my = lax.axis_index("i")
        left = lax.rem(my + (N_DEV - 1), N_DEV)
        right = lax.rem(my + 1, N_DEV)

        barrier = pltpu.get_barrier_semaphore()
        for nbr in (left, right):
            pl.semaphore_signal(barrier, inc=1, device_id=(nbr,),
                                device_id_type=pl.DeviceIdType.MESH)
        pl.semaphore_wait(barrier, 2)

        ksend[0] = k_ref[:, :, :WIN, :]
        ksend[1] = k_ref[:, :, S - WIN:, :]
        vsend[0] = v_ref[:, :, :WIN, :]
        vsend[1] = v_ref[:, :, S - WIN:, :]

        to_left, to_right = [], []
        for idx, (sbuf, dst_buf) in enumerate([(ksend, kbuf), (vsend, vbuf)]):
            r_l = pltpu.make_async_remote_copy(
                src_ref=sbuf.at[0],
                dst_ref=dst_buf.at[:, :, pl.ds(WIN + S, WIN), :],
                send_sem=send_sems.at[2 * idx],
                recv_sem=recv_sems.at[2 * idx],
                device_id=(left,), device_id_type=pl.DeviceIdType.MESH)
            r_r = pltpu.make_async_remote_copy(
                src_ref=sbuf.at[1],
                dst_ref=dst_buf.at[:, :, pl.ds(0, WIN), :],
                send_sem=send_sems.at[2 * idx + 1],
                recv_sem=recv_sems.at[2 * idx + 1],
                device_id=(right,), device_id_type=pl.DeviceIdType.MESH)
            r_l.start()
            r_r.start()
            to_left.append(r_l)
            to_right.append(r_r)

        kbuf[:, :, WIN:WIN + S] = k_ref[...]
        vbuf[:, :, WIN:WIN + S] = v_ref[...]

        wq = wq_ref[...]
        q = [jnp.dot(x_ref[b], wq, preferred_element_type=F32).astype(BF)
             for b in range(B)]

        ii = lax.broadcasted_iota(jnp.int32, (QB, KB), 0)
        jj = lax.broadcasted_iota(jnp.int32, (QB, KB), 1)
        band = (jj >= ii) & (jj <= ii + 2 * WIN)
        masks = []
        for t in range(NQB):
            j_glob = jj + t * QB + my * S - WIN
            masks.append(band & (j_glob >= 0) & (j_glob < N_DEV * S))

        ones_col = jnp.ones((KB, 1), BF)
        neg = jnp.float32(-1e9).astype(BF)
        wo = wo_ref[...]

        def block(b, t):
            cs = []
            for h in range(H):
                qt = q[b][t * QB:(t + 1) * QB, h * D:(h + 1) * D]
                kt = kbuf[b, h, t * QB:t * QB + KB, :]
                s = lax.dot_general(qt, kt, (((1,), (1,)), ((), ())),
                                    preferred_element_type=F32)
                w = jnp.exp(jnp.where(masks[t], s, -1e9)).astype(BF)
                denom = jnp.dot(w, ones_col, preferred_element_type=F32)
                c = jnp.dot(w, vbuf[b, h, t * QB:t * QB + KB, :],
                            preferred_element_type=F32)
                cs.append(c / denom)
            ctx = jnp.concatenate(cs, axis=1).astype(BF)
            out_ref[b, t * QB:(t + 1) * QB] = jnp.dot(
                ctx, wo, preferred_element_type=F32)

        for b in range(B):
            block(b, 1)
            block(b, 2)
        for r in to_right:
            r.wait_recv()
        for b in range(B):
            block(b, 0)
        for r in to_left:
            r.wait_recv()
        for b in range(B):
            block(b, 3)

        for r in to_left + to_right:
            r.wait_send()

    return pl.pallas_call(
        body,
        out_shape=jax.ShapeDtypeStruct((B, S, 768), F32),
        in_specs=[pl.BlockSpec(memory_space=pltpu.VMEM)] * 5,
        out_specs=pl.BlockSpec(memory_space=pltpu.VMEM),
        scratch_shapes=[
            pltpu.VMEM((B, H, E, D), BF),
            pltpu.VMEM((B, H, E, D), BF),
            pltpu.VMEM((2, B, H, WIN, D), BF),
            pltpu.VMEM((2, B, H, WIN, D), BF),
            pltpu.SemaphoreType.DMA((4,)),
            pltpu.SemaphoreType.DMA((4,)),
        ],
        compiler_params=pltpu.CompilerParams(collective_id=0),
    )(x_bf, wq_bf, k_bf, v_bf, wo_bf)


# device time: 34149 ns/iter; 1.5539x vs baseline; 1.0025x over previous
import jax
import jax.numpy as jnp
from jax import lax
from jax.experimental import pallas as pl
from jax.experimental.pallas import tpu as pltpu

N_DEV = 8
B = 2
S = 512
H = 8
D = 64
WIN = 128
E = S + 2 * WIN
QB = 128
KB = QB + 2 * WIN
NQB = S // QB
BF = jnp.bfloat16
F32 = jnp.float32


def kernel(x, Wq, K_ext, V_ext, Wo):
    x_bf = x.astype(BF)
    wq_bf = (Wq * 0.125).astype(BF)
    wo_bf = Wo.astype(BF)
    k_bf = jnp.transpose(K_ext, (0, 2, 1, 3)).astype(BF)
    v_bf = jnp.transpose(V_ext, (0, 2, 1, 3)).astype(BF)

    def body(x_ref, wq_ref, k_ref, v_ref, wo_ref, out_ref,
             kbuf, vbuf, ksend, vsend, send_sems, recv_sems):
        my = lax.axis_index("i")
        left = lax.rem(my + (N_DEV - 1), N_DEV)
        right = lax.rem(my + 1, N_DEV)

        barrier = pltpu.get_barrier_semaphore()
        for nbr in (left, right):
            pl.semaphore_signal(barrier, inc=1, device_id=(nbr,),
                                device_id_type=pl.DeviceIdType.MESH)
        pl.semaphore_wait(barrier, 2)

        ksend[0] = k_ref[:, :, :WIN, :]
        ksend[1] = k_ref[:, :, S - WIN:, :]
        vsend[0] = v_ref[:, :, :WIN, :]
        vsend[1] = v_ref[:, :, S - WIN:, :]

        to_left, to_right = [], []
        for idx, (sbuf, dst_buf) in enumerate([(ksend, kbuf), (vsend, vbuf)]):
            r_l = pltpu.make_async_remote_copy(
                src_ref=sbuf.at[0],
                dst_ref=dst_buf.at[:, :, pl.ds(WIN + S, WIN), :],
                send_sem=send_sems.at[2 * idx],
                recv_sem=recv_sems.at[2 * idx],
                device_id=(left,), device_id_type=pl.DeviceIdType.MESH)
            r_r = pltpu.make_async_remote_copy(
                src_ref=sbuf.at[1],
                dst_ref=dst_buf.at[:, :, pl.ds(0, WIN), :],
                send_sem=send_sems.at[2 * idx + 1],
                recv_sem=recv_sems.at[2 * idx + 1],
                device_id=(right,), device_id_type=pl.DeviceIdType.MESH)
            r_l.start()
            r_r.start()
            to_left.append(r_l)
            to_right.append(r_r)

        kbuf[:, :, WIN:WIN + S] = k_ref[...]
        vbuf[:, :, WIN:WIN + S] = v_ref[...]

        wq = wq_ref[...]
        q = [jnp.dot(x_ref[b], wq, preferred_element_type=F32).astype(BF)
             for b in range(B)]

        ii = lax.broadcasted_iota(jnp.int32, (QB, KB), 0)
        jj = lax.broadcasted_iota(jnp.int32, (QB, KB), 1)
        band = (jj >= ii) & (jj <= ii + 2 * WIN)
        masks = []
        for t in range(NQB):
            j_glob = jj + t * QB + my * S - WIN
            masks.append(band & (j_glob >= 0) & (j_glob < N_DEV * S))

        ones_col = jnp.ones((KB, 1), BF)
        neg = jnp.float32(-1e9).astype(BF)
        wo = wo_ref[...]

        def block(b, t):
            cs = []
            for h in range(H):
                qt = q[b][t * QB:(t + 1) * QB, h * D:(h + 1) * D]
                kt = kbuf[b, h, t * QB:t * QB + KB, :]
                s = lax.dot_general(qt, kt, (((1,), (1,)), ((), ())),
                                    preferred_element_type=F32).astype(BF)
                w = jnp.exp(jnp.where(masks[t], s, neg))
                denom = jnp.dot(w, ones_col, preferred_element_type=F32)
                c = jnp.dot(w, vbuf[b, h, t * QB:t * QB + KB, :],
                            preferred_element_type=F32)
                cs.append(c / denom)
            ctx = jnp.concatenate(cs, axis=1).astype(BF)
            out_ref[b, t * QB:(t + 1) * QB] = jnp.dot(
                ctx, wo, preferred_element_type=F32)

        for b in range(B):
            block(b, 1)
            block(b, 2)
        for r in to_right:
            r.wait_recv()
        for b in range(B):
            block(b, 0)
        for r in to_left:
            r.wait_recv()
        for b in range(B):
            block(b, 3)

        for r in to_left + to_right:
            r.wait_send()

    return pl.pallas_call(
        body,
        out_shape=jax.ShapeDtypeStruct((B, S, 768), F32),
        in_specs=[pl.BlockSpec(memory_space=pltpu.VMEM)] * 5,
        out_specs=pl.BlockSpec(memory_space=pltpu.VMEM),
        scratch_shapes=[
            pltpu.VMEM((B, H, E, D), BF),
            pltpu.VMEM((B, H, E, D), BF),
            pltpu.VMEM((2, B, H, WIN, D), BF),
            pltpu.VMEM((2, B, H, WIN, D), BF),
            pltpu.SemaphoreType.DMA((4,)),
            pltpu.SemaphoreType.DMA((4,)),
        ],
        compiler_params=pltpu.CompilerParams(collective_id=0),
    )(x_bf, wq_bf, k_bf, v_bf, wo_bf)
